# Initial kernel scaffold; baseline (speedup 1.0000x reference)
import jax
import jax.numpy as jnp
from jax.experimental import pallas as pl


def kernel(x, expert_sel, w1, b1, w2, b2):
    raise NotImplementedError("write your pallas kernel here")


# speedup vs baseline: 1.7624x; 1.7624x over previous
import jax
import jax.numpy as jnp
from jax.experimental import pallas as pl
from jax.experimental.pallas import tpu as pltpu

_E = 8
_BT = 512


def _router_body(x_ref, sw_ref, gate_ref):
    xb = x_ref[...]
    logits = jax.lax.dot_general(
        xb, sw_ref[...], (((1,), (1,)), ((), ())),
        preferred_element_type=jnp.float32)
    eidx = jax.lax.broadcasted_iota(jnp.int32, logits.shape, 1)
    krank = jnp.zeros_like(logits)
    for j in range(_E):
        vj = logits[:, j:j + 1]
        beats = (vj > logits) | ((vj == logits) & (j < eidx))
        krank += beats.astype(jnp.float32)
    gate_ref[...] = jnp.where(krank < 2.0, jax.nn.sigmoid(logits), 0.0)


def _dense_body(gate_ref, x_ref, w1_ref, b1_ref, w2_ref, b2_ref, o_ref):
    e = pl.program_id(1)

    @pl.when(e == 0)
    def _init():
        o_ref[...] = jnp.zeros_like(o_ref)

    xb = x_ref[...]
    h = jax.lax.dot_general(
        xb, w1_ref[0], (((1,), (1,)), ((), ())),
        preferred_element_type=jnp.float32)
    h = jnp.maximum(h + b1_ref[0], 0.0)
    y = jax.lax.dot_general(
        h, w2_ref[0], (((1,), (1,)), ((), ())),
        preferred_element_type=jnp.float32)
    y = y + b2_ref[0]
    onehot = (jax.lax.broadcasted_iota(jnp.int32, (1, _E), 1) == e)
    ge = jnp.sum(gate_ref[...] * onehot.astype(jnp.float32), axis=1,
                 keepdims=True)
    o_ref[...] += ge * y


def kernel(x, expert_sel, w1, b1, w2, b2):
    B, S, D = x.shape
    T = B * S
    H = w1.shape[1]
    xt = x.reshape(T, D)

    gate = pl.pallas_call(
        _router_body,
        grid=(T // _BT,),
        in_specs=[
            pl.BlockSpec((_BT, D), lambda i: (i, 0)),
            pl.BlockSpec((_E, D), lambda i: (0, 0)),
        ],
        out_specs=pl.BlockSpec((_BT, _E), lambda i: (i, 0)),
        out_shape=jax.ShapeDtypeStruct((T, _E), jnp.float32),
    )(xt, expert_sel)

    out = pl.pallas_call(
        _dense_body,
        grid=(T // _BT, _E),
        in_specs=[
            pl.BlockSpec((_BT, _E), lambda i, e: (i, 0)),
            pl.BlockSpec((_BT, D), lambda i, e: (i, 0)),
            pl.BlockSpec((1, H, D), lambda i, e: (e, 0, 0)),
            pl.BlockSpec((1, 1, H), lambda i, e: (e, 0, 0)),
            pl.BlockSpec((1, D, H), lambda i, e: (e, 0, 0)),
            pl.BlockSpec((1, 1, D), lambda i, e: (e, 0, 0)),
        ],
        out_specs=pl.BlockSpec((_BT, D), lambda i, e: (i, 0)),
        out_shape=jax.ShapeDtypeStruct((T, D), jnp.float32),
    )(gate, xt, w1, b1.reshape(_E, 1, H), w2, b2.reshape(_E, 1, D))

    return out.reshape(B, S, D), jnp.array(0.0, dtype=x.dtype)


# speedup vs baseline: 2.3793x; 1.3501x over previous
import jax
import jax.numpy as jnp
from jax.experimental import pallas as pl
from jax.experimental.pallas import tpu as pltpu

_E = 8
_BT = 1024


def _router_body(x_ref, sw_ref, gate_ref):
    xb = x_ref[...]
    logits = jax.lax.dot_general(
        xb, sw_ref[...], (((1,), (1,)), ((), ())),
        preferred_element_type=jnp.float32)
    eidx = jax.lax.broadcasted_iota(jnp.int32, logits.shape, 1)
    krank = jnp.zeros_like(logits)
    for j in range(_E):
        vj = logits[:, j:j + 1]
        beats = (vj > logits) | ((vj == logits) & (j < eidx))
        krank += beats.astype(jnp.float32)
    gate_ref[...] = jnp.where(krank < 2.0, jax.nn.sigmoid(logits), 0.0)


def _dense_body(gate_ref, x_ref, w1_ref, b1_ref, w2_ref, b2_ref, o_ref):
    e = pl.program_id(1)

    @pl.when(e == 0)
    def _init():
        o_ref[...] = jnp.zeros_like(o_ref)

    xb = x_ref[...].astype(jnp.bfloat16)
    h = jax.lax.dot_general(
        xb, w1_ref[0].astype(jnp.bfloat16), (((1,), (1,)), ((), ())),
        preferred_element_type=jnp.float32)
    h = jnp.maximum(h + b1_ref[0], 0.0).astype(jnp.bfloat16)
    y = jax.lax.dot_general(
        h, w2_ref[0].astype(jnp.bfloat16), (((1,), (1,)), ((), ())),
        preferred_element_type=jnp.float32)
    y = y + b2_ref[0]
    onehot = (jax.lax.broadcasted_iota(jnp.int32, (1, _E), 1) == e)
    ge = jnp.sum(gate_ref[...] * onehot.astype(jnp.float32), axis=1,
                 keepdims=True)
    o_ref[...] += ge * y


def kernel(x, expert_sel, w1, b1, w2, b2):
    B, S, D = x.shape
    T = B * S
    H = w1.shape[1]
    xt = x.reshape(T, D)

    gate = pl.pallas_call(
        _router_body,
        grid=(T // _BT,),
        in_specs=[
            pl.BlockSpec((_BT, D), lambda i: (i, 0)),
            pl.BlockSpec((_E, D), lambda i: (0, 0)),
        ],
        out_specs=pl.BlockSpec((_BT, _E), lambda i: (i, 0)),
        out_shape=jax.ShapeDtypeStruct((T, _E), jnp.float32),
    )(xt, expert_sel)

    out = pl.pallas_call(
        _dense_body,
        grid=(T // _BT, _E),
        in_specs=[
            pl.BlockSpec((_BT, _E), lambda i, e: (i, 0)),
            pl.BlockSpec((_BT, D), lambda i, e: (i, 0)),
            pl.BlockSpec((1, H, D), lambda i, e: (e, 0, 0)),
            pl.BlockSpec((1, 1, H), lambda i, e: (e, 0, 0)),
            pl.BlockSpec((1, D, H), lambda i, e: (e, 0, 0)),
            pl.BlockSpec((1, 1, D), lambda i, e: (e, 0, 0)),
        ],
        out_specs=pl.BlockSpec((_BT, D), lambda i, e: (i, 0)),
        out_shape=jax.ShapeDtypeStruct((T, D), jnp.float32),
    )(gate, xt, w1, b1.reshape(_E, 1, H), w2, b2.reshape(_E, 1, D))

    return out.reshape(B, S, D), jnp.array(0.0, dtype=x.dtype)
